# baseline (speedup 1.0000x reference)
import jax
import jax.numpy as jnp
from jax.experimental import pallas as pl
from jax.experimental.pallas import tpu as pltpu

_B, _N = 128, 32
_HIDDEN = 64
_LATENT = 128
_DIMS = [(64, 128), (128, 256), (256, 512)]
_CB = 16
_R = _CB * _N


def _silu(x):
    u = 0.5 * x
    return u * jnp.tanh(u) + u


def _body(types_ref, pos_ref, lat_ref, emb_ref, posW_ref, posb_ref,
          latW_ref, latb_ref, *rest):
    refs = list(rest)
    out_ref = refs.pop()
    f32 = jnp.float32

    types = types_ref[...]
    posf = pos_ref[...]
    latf = lat_ref[...]

    iota = jax.lax.broadcasted_iota(jnp.int32, (_R, 128), 1)
    oh = (types == iota).astype(f32)
    h = jnp.dot(oh, emb_ref[...], preferred_element_type=f32)
    h = h + jnp.dot(posf, posW_ref[...], preferred_element_type=f32) + posb_ref[...]
    h = h + jnp.dot(latf, latW_ref[...], preferred_element_type=f32) + latb_ref[...]

    p = posf.reshape(_CB, _N, 3)
    sq = jnp.sum(p * p, axis=-1, keepdims=True)
    one = jnp.ones_like(sq)
    paug = jnp.concatenate([p, sq, one], axis=-1)
    qaug = jnp.concatenate([-2.0 * p, one, sq], axis=-1)
    d2 = jax.lax.dot_general(paug, qaug, (((2,), (2,)), ((0,), (0,))),
                             preferred_element_type=f32)
    ii = jax.lax.broadcasted_iota(jnp.int32, (_N, _N), 0)
    jj = jax.lax.broadcasted_iota(jnp.int32, (_N, _N), 1)
    d2 = jnp.where((ii == jj)[None], 0.0, jnp.maximum(d2, 0.0))
    dist = jnp.sqrt(d2 + 1e-8)[..., None]
    c_self = jnp.sqrt(f32(1e-8))

    for l, (din, dout) in enumerate(_DIMS):
        mWh, mWr, mWd, mb, uWh, uWa, ub = refs[7 * l:7 * l + 7]
        hp2 = 0.5 * (jnp.dot(h, mWh[...], preferred_element_type=f32)
                     + mb[...])
        A2 = 0.5 * jnp.dot(posf, mWr[...], preferred_element_type=f32)
        wd2 = 0.5 * mWd[...]
        hpA = hp2 + A2
        bf = jnp.bfloat16
        u = (hpA.astype(bf).reshape(_CB, _N, 1, dout)
             - A2.astype(bf).reshape(_CB, 1, _N, dout)
             + dist.astype(bf) * wd2.astype(bf).reshape(1, 1, 1, dout))
        m = u * jnp.tanh(u)
        ones_s = jnp.ones((_CB, _N), jnp.bfloat16)
        aggm = jax.lax.dot_general(ones_s, m, (((1,), (1,)), ((0,), (0,))),
                                   preferred_element_type=f32)
        sum_hpA = jnp.sum(hpA.reshape(_CB, _N, dout), axis=1, keepdims=True)
        sum_dist = jnp.sum(dist, axis=1)
        sumu = (sum_hpA - f32(_N) * A2.reshape(_CB, _N, dout)
                + sum_dist * wd2.reshape(1, 1, dout))
        agg = (aggm + sumu).reshape(_R, dout)
        us = hp2 + c_self * wd2
        agg = agg - (us * jnp.tanh(us) + us)
        h = _silu(jnp.dot(h, uWh[...], preferred_element_type=f32)
                        + jnp.dot(agg, uWa[...], preferred_element_type=f32)
                        + ub[...])

    g = jnp.mean(h.reshape(_CB, _N, h.shape[-1]), axis=1)
    p1W, p1b, p2W, p2b = refs[21:25]
    t = _silu(jnp.dot(g, p1W[...], preferred_element_type=f32) + p1b[...])
    out_ref[...] = jnp.dot(t, p2W[...], preferred_element_type=f32) + p2b[...]


def kernel(atom_types, positions, lattice, atom_emb, pos_W, pos_b, lat_W,
           lat_b, msg_W0, msg_b0, upd_W0, upd_b0, msg_W1, msg_b1, upd_W1,
           upd_b1, msg_W2, msg_b2, upd_W2, upd_b2, proj1_W, proj1_b,
           proj2_W, proj2_b):
    f32 = jnp.float32
    types = atom_types.astype(jnp.int32).reshape(_B * _N, 1)
    posf = positions.reshape(_B * _N, 3)
    latf = jnp.broadcast_to(lattice[:, None, :], (_B, _N, 6)).reshape(_B * _N, 6)
    emb = jnp.zeros((128, _HIDDEN), f32).at[:atom_emb.shape[0]].set(atom_emb)

    args = [types, posf, latf, emb, pos_W, pos_b.reshape(1, -1), lat_W,
            lat_b.reshape(1, -1)]
    for (din, dout), mW, mb, uW, ub in zip(
            _DIMS,
            (msg_W0, msg_W1, msg_W2), (msg_b0, msg_b1, msg_b2),
            (upd_W0, upd_W1, upd_W2), (upd_b0, upd_b1, upd_b2)):
        args += [mW[:din], mW[din:din + 3], mW[din + 3:din + 4],
                 mb.reshape(1, -1), uW[:din], uW[din:], ub.reshape(1, -1)]
    args += [proj1_W, proj1_b.reshape(1, -1), proj2_W, proj2_b.reshape(1, -1)]

    return _single_call(*args)


def _single_call(*args):
    f32 = jnp.float32
    rows = args[0].shape[0]
    b_local = rows // _N
    grid = (b_local // _CB,)
    in_specs = [
        pl.BlockSpec((_R, 1), lambda i: (i, 0)),
        pl.BlockSpec((_R, 3), lambda i: (i, 0)),
        pl.BlockSpec((_R, 6), lambda i: (i, 0)),
    ] + [pl.BlockSpec(a.shape, lambda i: (0,) * a.ndim) for a in args[3:]]

    return pl.pallas_call(
        _body,
        grid=grid,
        in_specs=in_specs,
        out_specs=pl.BlockSpec((_CB, _LATENT), lambda i: (i, 0)),
        out_shape=jax.ShapeDtypeStruct((b_local, _LATENT), f32),
        compiler_params=pltpu.CompilerParams(
            dimension_semantics=("parallel",)),
    )(*args)


# speedup vs baseline: 1.8135x; 1.8135x over previous
import jax
import jax.numpy as jnp
from jax.experimental import pallas as pl
from jax.experimental.pallas import tpu as pltpu

_B, _N = 128, 32
_HIDDEN = 64
_LATENT = 128
_DIMS = [(64, 128), (128, 256), (256, 512)]
_CB = 16
_R = _CB * _N


def _silu(x):
    u = 0.5 * x
    return u * jnp.tanh(u) + u


def _body(types_ref, pos_ref, lat_ref, emb_ref, posW_ref, posb_ref,
          latW_ref, latb_ref, *rest):
    refs = list(rest)
    out_ref = refs.pop()
    f32 = jnp.float32

    types = types_ref[...]
    posf = pos_ref[...]
    latf = lat_ref[...]

    iota = jax.lax.broadcasted_iota(jnp.int32, (_R, 128), 1)
    oh = (types == iota).astype(f32)
    h = jnp.dot(oh, emb_ref[...], preferred_element_type=f32)
    h = h + jnp.dot(posf, posW_ref[...], preferred_element_type=f32) + posb_ref[...]
    h = h + jnp.dot(latf, latW_ref[...], preferred_element_type=f32) + latb_ref[...]

    p = posf.reshape(_CB, _N, 3)
    sq = jnp.sum(p * p, axis=-1, keepdims=True)
    one = jnp.ones_like(sq)
    paug = jnp.concatenate([p, sq, one], axis=-1)
    qaug = jnp.concatenate([-2.0 * p, one, sq], axis=-1)
    d2 = jax.lax.dot_general(paug, qaug, (((2,), (2,)), ((0,), (0,))),
                             preferred_element_type=f32)
    ii = jax.lax.broadcasted_iota(jnp.int32, (_N, _N), 0)
    jj = jax.lax.broadcasted_iota(jnp.int32, (_N, _N), 1)
    d2 = jnp.where((ii == jj)[None], 0.0, jnp.maximum(d2, 0.0))
    dist = jnp.sqrt(d2 + 1e-8)[..., None]
    c_self = jnp.sqrt(f32(1e-8))

    bf = jnp.bfloat16
    lane = jax.lax.broadcasted_iota(jnp.int32, (_N, _N, 128), 2)
    srow = jax.lax.broadcasted_iota(jnp.int32, (_N, _N, 128), 0)
    drow = jax.lax.broadcasted_iota(jnp.int32, (_N, _N, 128), 1)
    cbase = ((lane == srow) | (lane == drow + _N)).astype(bf)
    c4 = jnp.where(lane[None] == 2 * _N, dist.astype(bf), cbase[None])
    cmat = c4.reshape(_CB, _N * _N, 128)

    for l, (din, dout) in enumerate(_DIMS):
        mWh, mWr, mWd, mb, uWh, uWa, ub = refs[7 * l:7 * l + 7]
        hp2 = 0.5 * (jnp.dot(h, mWh[...], preferred_element_type=f32)
                     + mb[...])
        A2 = 0.5 * jnp.dot(posf, mWr[...], preferred_element_type=f32)
        wd2 = 0.5 * mWd[...]
        hpA = hp2 + A2
        W = jnp.concatenate(
            [hpA.reshape(_CB, _N, dout), -A2.reshape(_CB, _N, dout),
             jnp.broadcast_to(wd2, (_CB, 1, dout)),
             jnp.zeros((_CB, 128 - 2 * _N - 1, dout), f32)],
            axis=1).astype(bf)
        u = jax.lax.dot_general(cmat, W, (((2,), (1,)), ((0,), (0,))),
                                preferred_element_type=f32)
        m = u * jnp.tanh(u)
        aggm = jnp.sum(m.reshape(_CB, _N, _N, dout), axis=1)
        sum_hpA = jnp.sum(hpA.reshape(_CB, _N, dout), axis=1, keepdims=True)
        sum_dist = jnp.sum(dist, axis=1)
        sumu = (sum_hpA - f32(_N) * A2.reshape(_CB, _N, dout)
                + sum_dist * wd2.reshape(1, 1, dout))
        agg = (aggm + sumu).reshape(_R, dout)
        us = hp2 + c_self * wd2
        agg = agg - (us * jnp.tanh(us) + us)
        h = _silu(jnp.dot(h, uWh[...], preferred_element_type=f32)
                        + jnp.dot(agg, uWa[...], preferred_element_type=f32)
                        + ub[...])

    g = jnp.mean(h.reshape(_CB, _N, h.shape[-1]), axis=1)
    p1W, p1b, p2W, p2b = refs[21:25]
    t = _silu(jnp.dot(g, p1W[...], preferred_element_type=f32) + p1b[...])
    out_ref[...] = jnp.dot(t, p2W[...], preferred_element_type=f32) + p2b[...]


def kernel(atom_types, positions, lattice, atom_emb, pos_W, pos_b, lat_W,
           lat_b, msg_W0, msg_b0, upd_W0, upd_b0, msg_W1, msg_b1, upd_W1,
           upd_b1, msg_W2, msg_b2, upd_W2, upd_b2, proj1_W, proj1_b,
           proj2_W, proj2_b):
    f32 = jnp.float32
    types = atom_types.astype(jnp.int32).reshape(_B * _N, 1)
    posf = positions.reshape(_B * _N, 3)
    latf = jnp.broadcast_to(lattice[:, None, :], (_B, _N, 6)).reshape(_B * _N, 6)
    emb = jnp.zeros((128, _HIDDEN), f32).at[:atom_emb.shape[0]].set(atom_emb)

    args = [types, posf, latf, emb, pos_W, pos_b.reshape(1, -1), lat_W,
            lat_b.reshape(1, -1)]
    for (din, dout), mW, mb, uW, ub in zip(
            _DIMS,
            (msg_W0, msg_W1, msg_W2), (msg_b0, msg_b1, msg_b2),
            (upd_W0, upd_W1, upd_W2), (upd_b0, upd_b1, upd_b2)):
        args += [mW[:din], mW[din:din + 3], mW[din + 3:din + 4],
                 mb.reshape(1, -1), uW[:din], uW[din:], ub.reshape(1, -1)]
    args += [proj1_W, proj1_b.reshape(1, -1), proj2_W, proj2_b.reshape(1, -1)]

    return _single_call(*args)


def _single_call(*args):
    f32 = jnp.float32
    rows = args[0].shape[0]
    b_local = rows // _N
    grid = (b_local // _CB,)
    in_specs = [
        pl.BlockSpec((_R, 1), lambda i: (i, 0)),
        pl.BlockSpec((_R, 3), lambda i: (i, 0)),
        pl.BlockSpec((_R, 6), lambda i: (i, 0)),
    ] + [pl.BlockSpec(a.shape, lambda i: (0,) * a.ndim) for a in args[3:]]

    return pl.pallas_call(
        _body,
        grid=grid,
        in_specs=in_specs,
        out_specs=pl.BlockSpec((_CB, _LATENT), lambda i: (i, 0)),
        out_shape=jax.ShapeDtypeStruct((b_local, _LATENT), f32),
        compiler_params=pltpu.CompilerParams(
            dimension_semantics=("parallel",)),
    )(*args)
